# initial kernel scaffold (unmeasured)
import jax
import jax.numpy as jnp
from jax import lax
from jax.experimental import pallas as pl
from jax.experimental.pallas import tpu as pltpu

N_DEV = 8
SQ = 2048
D = 1024
HQ = 8
DH = 128
SKV_LOC = 2048
BLK = 64
CHUNK = SQ // N_DEV
QT = 512
N_QT = SQ // QT
SCALE = 0.08838834764831843
NEG = -1e9


def kernel(x, Wq, K_ext, V_ext, Wo):
    def body(x_ref, wq_ref, k_ref, v_ref, wo_ref, out_ref,
             q_scr, o_acc, ml_acc, o_rs, ml_rs,
             rs_send_sems, rs_recv_sems, ag_send_sems, ag_recv_sems):
        p = lax.axis_index("i")
        left = lax.rem(p - 1 + N_DEV, N_DEV)
        right = lax.rem(p + 1, N_DEV)

        barrier = pltpu.get_barrier_semaphore()
        for nbr in (left, right):
            pl.semaphore_signal(barrier, inc=1, device_id=(nbr,),
                                device_id_type=pl.DeviceIdType.MESH)
        pl.semaphore_wait(barrier, 2)

        q_scr[:, :] = jnp.dot(x_ref[0], wq_ref[:, :],
                              preferred_element_type=jnp.float32)

        kb = p * (SKV_LOC // BLK) + lax.broadcasted_iota(
            jnp.int32, (1, SKV_LOC), 1) // BLK
        for h in range(HQ):
            kh = k_ref[0, :, h, :]
            vh = v_ref[0, :, h, :]
            for t in range(N_QT):
                qt = q_scr[t * QT:(t + 1) * QT, h * DH:(h + 1) * DH]
                s = lax.dot_general(
                    qt, kh, (((1,), (1,)), ((), ())),
                    preferred_element_type=jnp.float32) * SCALE
                qb = (t * QT + lax.broadcasted_iota(
                    jnp.int32, (QT, 1), 0)) // BLK
                mask = (qb == kb) | (kb == 0) | (lax.rem(qb + kb, 3) == 0)
                s = jnp.where(mask, s, NEG)
                m = jnp.max(s, axis=1, keepdims=True)
                w = jnp.exp(s - m)
                lsum = jnp.sum(w, axis=1, keepdims=True)
                o = lax.dot_general(
                    w, vh, (((1,), (0,)), ((), ())),
                    preferred_element_type=jnp.float32)
                o_acc[t * QT:(t + 1) * QT, h * DH:(h + 1) * DH] = o
                ml_acc[0, t * QT:(t + 1) * QT, h:h + 1] = m
                ml_acc[1, t * QT:(t + 1) * QT, h:h + 1] = lsum

        rdmas = []
        for s_hop in range(N_DEV - 1):
            c_send = lax.rem(p - s_hop + N_DEV, N_DEV)
            c_recv = lax.rem(p - s_hop - 1 + N_DEV, N_DEV)
            o_rdma = pltpu.make_async_remote_copy(
                src_ref=o_acc.at[pl.ds(c_send * CHUNK, CHUNK), :],
                dst_ref=o_rs.at[s_hop],
                send_sem=rs_send_sems.at[0, s_hop],
                recv_sem=rs_recv_sems.at[0, s_hop],
                device_id=(right,), device_id_type=pl.DeviceIdType.MESH)
            ml_rdma = pltpu.make_async_remote_copy(
                src_ref=ml_acc.at[:, pl.ds(c_send * CHUNK, CHUNK), :],
                dst_ref=ml_rs.at[s_hop],
                send_sem=rs_send_sems.at[1, s_hop],
                recv_sem=rs_recv_sems.at[1, s_hop],
                device_id=(right,), device_id_type=pl.DeviceIdType.MESH)
            o_rdma.start()
            ml_rdma.start()
            o_rdma.wait_recv()
            ml_rdma.wait_recv()
            rdmas += [o_rdma, ml_rdma]

            rows = pl.ds(c_recv * CHUNK, CHUNK)
            m_loc = ml_acc[0, rows, :]
            l_loc = ml_acc[1, rows, :]
            m_rx = ml_rs[s_hop, 0]
            l_rx = ml_rs[s_hop, 1]
            m_new = jnp.maximum(m_loc, m_rx)
            a_loc = jnp.exp(m_loc - m_new)
            a_rx = jnp.exp(m_rx - m_new)
            for h in range(HQ):
                cols = slice(h * DH, (h + 1) * DH)
                o_acc[rows, cols] = (
                    o_acc[rows, cols] * a_loc[:, h:h + 1]
                    + o_rs[s_hop][:, cols] * a_rx[:, h:h + 1])
            ml_acc[0, rows, :] = m_new
            ml_acc[1, rows, :] = l_loc * a_loc + l_rx * a_rx

        c_own = lax.rem(p + 1, N_DEV)
        rows = pl.ds(c_own * CHUNK, CHUNK)
        l_own = ml_acc[1, rows, :]
        ctx_cols = []
        for h in range(HQ):
            cols = slice(h * DH, (h + 1) * DH)
            ctx_cols.append(o_acc[rows, cols] / l_own[:, h:h + 1])
        ctx = jnp.concatenate(ctx_cols, axis=1)
        out_ref[0, rows, :] = jnp.dot(ctx, wo_ref[:, :],
                                      preferred_element_type=jnp.float32)

        for h_hop in range(N_DEV - 1):
            c_send = lax.rem(p + 1 - h_hop + N_DEV, N_DEV)
            src = out_ref.at[0, pl.ds(c_send * CHUNK, CHUNK), :]
            rdma = pltpu.make_async_remote_copy(
                src_ref=src, dst_ref=src,
                send_sem=ag_send_sems.at[h_hop],
                recv_sem=ag_recv_sems.at[h_hop],
                device_id=(right,), device_id_type=pl.DeviceIdType.MESH)
            rdma.start()
            rdma.wait_recv()
            rdmas.append(rdma)

        for r in rdmas:
            r.wait_send()

    return pl.pallas_call(
        body,
        out_shape=jax.ShapeDtypeStruct((1, SQ, D), jnp.float32),
        in_specs=[pl.BlockSpec(memory_space=pltpu.VMEM)] * 5,
        out_specs=pl.BlockSpec(memory_space=pltpu.VMEM),
        scratch_shapes=[
            pltpu.VMEM((SQ, D), jnp.float32),
            pltpu.VMEM((SQ, D), jnp.float32),
            pltpu.VMEM((2, SQ, HQ), jnp.float32),
            pltpu.VMEM((N_DEV - 1, CHUNK, D), jnp.float32),
            pltpu.VMEM((N_DEV - 1, 2, CHUNK, HQ), jnp.float32),
            pltpu.SemaphoreType.DMA((2, N_DEV - 1)),
            pltpu.SemaphoreType.DMA((2, N_DEV - 1)),
            pltpu.SemaphoreType.DMA((N_DEV - 1,)),
            pltpu.SemaphoreType.DMA((N_DEV - 1,)),
        ],
        compiler_params=pltpu.CompilerParams(collective_id=0),
    )(x, Wq, K_ext, V_ext, Wo)


# baseline (device time: 401309 ns/iter reference)
import jax
import jax.numpy as jnp
from jax import lax
from jax.experimental import pallas as pl
from jax.experimental.pallas import tpu as pltpu

N_DEV = 8
SQ = 2048
D = 1024
HQ = 8
DH = 128
SKV_LOC = 2048
BLK = 64
CHUNK = SQ // N_DEV
QT = 512
N_QT = SQ // QT
SCALE = 0.08838834764831843
NEG = -1e9


def _attn_body(x_ref, wq_ref, k_ref, v_ref, o_ref, m_ref, l_ref):
    p = lax.axis_index("i")
    t = pl.program_id(0)
    h = pl.program_id(1)
    q = jnp.dot(x_ref[0], wq_ref[:, :],
                preferred_element_type=jnp.float32)
    s = lax.dot_general(q, k_ref[:, :], (((1,), (1,)), ((), ())),
                        preferred_element_type=jnp.float32) * SCALE
    qb = (t * QT + lax.broadcasted_iota(jnp.int32, (QT, 1), 0)) // BLK
    kb = p * (SKV_LOC // BLK) + lax.broadcasted_iota(
        jnp.int32, (1, SKV_LOC), 1) // BLK
    mask = (qb == kb) | (kb == 0) | (lax.rem(qb + kb, 3) == 0)
    s = jnp.where(mask, s, NEG)
    m = jnp.max(s, axis=1, keepdims=True)
    w = jnp.exp(s - m)
    lsum = jnp.sum(w, axis=1, keepdims=True)
    o_ref[:, :] = lax.dot_general(w, v_ref[:, :],
                                  (((1,), (0,)), ((), ())),
                                  preferred_element_type=jnp.float32)
    lane = lax.broadcasted_iota(jnp.int32, (QT, HQ), 1)
    m_ref[:, :] = jnp.where(lane == h, m, m_ref[:, :])
    l_ref[:, :] = jnp.where(lane == h, lsum, l_ref[:, :])


def _ring_body(o_ref, m_ref, l_ref, wo_ref, out_ref,
               o_acc, ml_acc, o_rs, ml_rs,
               rs_send_sems, rs_recv_sems, ag_send_sems, ag_recv_sems):
    p = lax.axis_index("i")
    left = lax.rem(p - 1 + N_DEV, N_DEV)
    right = lax.rem(p + 1, N_DEV)

    barrier = pltpu.get_barrier_semaphore()
    for nbr in (left, right):
        pl.semaphore_signal(barrier, inc=1, device_id=(nbr,),
                            device_id_type=pl.DeviceIdType.MESH)
    pl.semaphore_wait(barrier, 2)

    o_acc[:, :] = o_ref[:, :]
    ml_acc[0, :, :] = m_ref[:, :]
    ml_acc[1, :, :] = l_ref[:, :]

    rdmas = []
    for s_hop in range(N_DEV - 1):
        c_send = lax.rem(p - s_hop + N_DEV, N_DEV)
        c_recv = lax.rem(p - s_hop - 1 + N_DEV, N_DEV)
        o_rdma = pltpu.make_async_remote_copy(
            src_ref=o_acc.at[pl.ds(c_send * CHUNK, CHUNK), :],
            dst_ref=o_rs.at[s_hop],
            send_sem=rs_send_sems.at[0, s_hop],
            recv_sem=rs_recv_sems.at[0, s_hop],
            device_id=(right,), device_id_type=pl.DeviceIdType.MESH)
        ml_rdma = pltpu.make_async_remote_copy(
            src_ref=ml_acc.at[:, pl.ds(c_send * CHUNK, CHUNK), :],
            dst_ref=ml_rs.at[s_hop],
            send_sem=rs_send_sems.at[1, s_hop],
            recv_sem=rs_recv_sems.at[1, s_hop],
            device_id=(right,), device_id_type=pl.DeviceIdType.MESH)
        o_rdma.start()
        ml_rdma.start()
        o_rdma.wait_recv()
        ml_rdma.wait_recv()
        rdmas += [o_rdma, ml_rdma]

        rows = pl.ds(c_recv * CHUNK, CHUNK)
        m_loc = ml_acc[0, rows, :]
        l_loc = ml_acc[1, rows, :]
        m_rx = ml_rs[s_hop, 0]
        l_rx = ml_rs[s_hop, 1]
        m_new = jnp.maximum(m_loc, m_rx)
        a_loc = jnp.exp(m_loc - m_new)
        a_rx = jnp.exp(m_rx - m_new)
        for h in range(HQ):
            cols = slice(h * DH, (h + 1) * DH)
            o_acc[rows, cols] = (
                o_acc[rows, cols] * a_loc[:, h:h + 1]
                + o_rs[s_hop][:, cols] * a_rx[:, h:h + 1])
        ml_acc[0, rows, :] = m_new
        ml_acc[1, rows, :] = l_loc * a_loc + l_rx * a_rx

    c_own = lax.rem(p + 1, N_DEV)
    rows = pl.ds(c_own * CHUNK, CHUNK)
    l_own = ml_acc[1, rows, :]
    ctx_cols = []
    for h in range(HQ):
        cols = slice(h * DH, (h + 1) * DH)
        ctx_cols.append(o_acc[rows, cols] / l_own[:, h:h + 1])
    ctx = jnp.concatenate(ctx_cols, axis=1)
    out_ref[0, rows, :] = jnp.dot(ctx, wo_ref[:, :],
                                  preferred_element_type=jnp.float32)

    for h_hop in range(N_DEV - 1):
        c_send = lax.rem(p + 1 - h_hop + N_DEV, N_DEV)
        src = out_ref.at[0, pl.ds(c_send * CHUNK, CHUNK), :]
        rdma = pltpu.make_async_remote_copy(
            src_ref=src, dst_ref=src,
            send_sem=ag_send_sems.at[h_hop],
            recv_sem=ag_recv_sems.at[h_hop],
            device_id=(right,), device_id_type=pl.DeviceIdType.MESH)
        rdma.start()
        rdma.wait_recv()
        rdmas.append(rdma)

    for r in rdmas:
        r.wait_send()


def kernel(x, Wq, K_ext, V_ext, Wo):
    k2 = K_ext.reshape(SKV_LOC, HQ * DH)
    v2 = V_ext.reshape(SKV_LOC, HQ * DH)
    o, m, l = pl.pallas_call(
        _attn_body,
        grid=(N_QT, HQ),
        in_specs=[
            pl.BlockSpec((1, QT, D), lambda t, h: (0, t, 0)),
            pl.BlockSpec((D, DH), lambda t, h: (0, h)),
            pl.BlockSpec((SKV_LOC, DH), lambda t, h: (0, h)),
            pl.BlockSpec((SKV_LOC, DH), lambda t, h: (0, h)),
        ],
        out_specs=[
            pl.BlockSpec((QT, DH), lambda t, h: (t, h)),
            pl.BlockSpec((QT, HQ), lambda t, h: (t, 0)),
            pl.BlockSpec((QT, HQ), lambda t, h: (t, 0)),
        ],
        out_shape=[
            jax.ShapeDtypeStruct((SQ, D), jnp.float32),
            jax.ShapeDtypeStruct((SQ, HQ), jnp.float32),
            jax.ShapeDtypeStruct((SQ, HQ), jnp.float32),
        ],
    )(x, Wq, k2, v2)

    return pl.pallas_call(
        _ring_body,
        out_shape=jax.ShapeDtypeStruct((1, SQ, D), jnp.float32),
        in_specs=[pl.BlockSpec(memory_space=pltpu.VMEM)] * 4,
        out_specs=pl.BlockSpec(memory_space=pltpu.VMEM),
        scratch_shapes=[
            pltpu.VMEM((SQ, D), jnp.float32),
            pltpu.VMEM((2, SQ, HQ), jnp.float32),
            pltpu.VMEM((N_DEV - 1, CHUNK, D), jnp.float32),
            pltpu.VMEM((N_DEV - 1, 2, CHUNK, HQ), jnp.float32),
            pltpu.SemaphoreType.DMA((2, N_DEV - 1)),
            pltpu.SemaphoreType.DMA((2, N_DEV - 1)),
            pltpu.SemaphoreType.DMA((N_DEV - 1,)),
            pltpu.SemaphoreType.DMA((N_DEV - 1,)),
        ],
        compiler_params=pltpu.CompilerParams(collective_id=0),
    )(o, m, l, Wo)


# device time: 294508 ns/iter; 1.3626x vs baseline; 1.3626x over previous
import jax
import jax.numpy as jnp
from jax import lax
from jax.experimental import pallas as pl
from jax.experimental.pallas import tpu as pltpu

N_DEV = 8
SQ = 2048
D = 1024
HQ = 8
DH = 128
SKV_LOC = 2048
BLK = 64
CHUNK = SQ // N_DEV
QT = 512
N_QT = SQ // QT
SCALE = 0.08838834764831843
NEG = -1e9


def _attn_body(x_ref, wq_ref, k_ref, v_ref, o_ref, m_ref, l_ref):
    p = lax.axis_index("i")
    t = pl.program_id(0)
    h = pl.program_id(1)
    q = jnp.dot(x_ref[0], wq_ref[:, :],
                preferred_element_type=jnp.float32)
    s = lax.dot_general(q, k_ref[:, :], (((1,), (1,)), ((), ())),
                        preferred_element_type=jnp.float32) * SCALE
    qb = (t * QT + lax.broadcasted_iota(jnp.int32, (QT, 1), 0)) // BLK
    kb = p * (SKV_LOC // BLK) + lax.broadcasted_iota(
        jnp.int32, (1, SKV_LOC), 1) // BLK
    mask = (qb == kb) | (kb == 0) | (lax.rem(qb + kb, 3) == 0)
    s = jnp.where(mask, s, NEG)
    m = jnp.max(s, axis=1, keepdims=True)
    w = jnp.exp(s - m)
    lsum = jnp.sum(w, axis=1, keepdims=True)
    o_ref[:, :] = lax.dot_general(w, v_ref[:, :],
                                  (((1,), (0,)), ((), ())),
                                  preferred_element_type=jnp.float32)
    lane = lax.broadcasted_iota(jnp.int32, (QT, HQ), 1)
    m_ref[:, :] = jnp.where(lane == h, m, m_ref[:, :])
    l_ref[:, :] = jnp.where(lane == h, lsum, l_ref[:, :])


HALF = CHUNK // 2


def _merge(o_acc, ml_acc, o_rx_ref, ml_rx_ref, rows):
    m_loc = ml_acc[0, rows, :]
    l_loc = ml_acc[1, rows, :]
    m_rx = ml_rx_ref[0]
    l_rx = ml_rx_ref[1]
    m_new = jnp.maximum(m_loc, m_rx)
    a_loc = jnp.exp(m_loc - m_new)
    a_rx = jnp.exp(m_rx - m_new)
    for h in range(HQ):
        cols = slice(h * DH, (h + 1) * DH)
        o_acc[rows, cols] = (
            o_acc[rows, cols] * a_loc[:, h:h + 1]
            + o_rx_ref[:, cols] * a_rx[:, h:h + 1])
    ml_acc[0, rows, :] = m_new
    ml_acc[1, rows, :] = l_loc * a_loc + l_rx * a_rx


def _ring_body(o_ref, m_ref, l_ref, wo_ref, out_ref,
               o_acc, ml_acc, o_cw, ml_cw, o_ccw, ml_ccw,
               cw_send_sems, cw_recv_sems, ccw_send_sems, ccw_recv_sems,
               ag_send_sems, ag_recv_sems):
    p = lax.axis_index("i")
    left = lax.rem(p - 1 + N_DEV, N_DEV)
    right = lax.rem(p + 1, N_DEV)

    barrier = pltpu.get_barrier_semaphore()
    for nbr in (left, right):
        pl.semaphore_signal(barrier, inc=1, device_id=(nbr,),
                            device_id_type=pl.DeviceIdType.MESH)
    pl.semaphore_wait(barrier, 2)

    o_acc[:, :] = o_ref[:, :]
    ml_acc[0, :, :] = m_ref[:, :]
    ml_acc[1, :, :] = l_ref[:, :]

    rdmas = []
    for s_hop in range(N_DEV - 1):
        cw_send = lax.rem(p - s_hop + N_DEV, N_DEV)
        cw_recv = lax.rem(p - s_hop - 1 + N_DEV, N_DEV)
        ccw_send = lax.rem(p + s_hop, N_DEV)
        ccw_recv = lax.rem(p + s_hop + 1, N_DEV)
        hop = []
        for (c_send, dev, o_rs, ml_rs, ssems, rsems, off) in (
                (cw_send, right, o_cw, ml_cw, cw_send_sems, cw_recv_sems, 0),
                (ccw_send, left, o_ccw, ml_ccw, ccw_send_sems, ccw_recv_sems,
                 HALF)):
            o_rdma = pltpu.make_async_remote_copy(
                src_ref=o_acc.at[pl.ds(c_send * CHUNK + off, HALF), :],
                dst_ref=o_rs.at[s_hop],
                send_sem=ssems.at[0, s_hop],
                recv_sem=rsems.at[0, s_hop],
                device_id=(dev,), device_id_type=pl.DeviceIdType.MESH)
            ml_rdma = pltpu.make_async_remote_copy(
                src_ref=ml_acc.at[:, pl.ds(c_send * CHUNK + off, HALF), :],
                dst_ref=ml_rs.at[s_hop],
                send_sem=ssems.at[1, s_hop],
                recv_sem=rsems.at[1, s_hop],
                device_id=(dev,), device_id_type=pl.DeviceIdType.MESH)
            o_rdma.start()
            ml_rdma.start()
            hop += [o_rdma, ml_rdma]
        rdmas += hop
        hop[0].wait_recv()
        hop[1].wait_recv()
        _merge(o_acc, ml_acc, o_cw.at[s_hop], ml_cw.at[s_hop],
               pl.ds(cw_recv * CHUNK, HALF))
        hop[2].wait_recv()
        hop[3].wait_recv()
        _merge(o_acc, ml_acc, o_ccw.at[s_hop], ml_ccw.at[s_hop],
               pl.ds(ccw_recv * CHUNK + HALF, HALF))

    for (c_own, off) in ((lax.rem(p + 1, N_DEV), 0),
                         (lax.rem(p - 1 + N_DEV, N_DEV), HALF)):
        rows = pl.ds(c_own * CHUNK + off, HALF)
        l_own = ml_acc[1, rows, :]
        ctx_cols = []
        for h in range(HQ):
            cols = slice(h * DH, (h + 1) * DH)
            ctx_cols.append(o_acc[rows, cols] / l_own[:, h:h + 1])
        ctx = jnp.concatenate(ctx_cols, axis=1)
        out_ref[0, rows, :] = jnp.dot(ctx, wo_ref[:, :],
                                      preferred_element_type=jnp.float32)

    for h_hop in range(N_DEV - 1):
        cw_c = lax.rem(p + 1 - h_hop + N_DEV, N_DEV)
        ccw_c = lax.rem(p - 1 + h_hop + N_DEV, N_DEV)
        hop = []
        for (c, dev, ssems, rsems, off) in (
                (cw_c, right, ag_send_sems, ag_recv_sems, 0),
                (ccw_c, left, ag_send_sems, ag_recv_sems, HALF)):
            src = out_ref.at[0, pl.ds(c * CHUNK + off, HALF), :]
            rdma = pltpu.make_async_remote_copy(
                src_ref=src, dst_ref=src,
                send_sem=ssems.at[0 if off == 0 else 1, h_hop],
                recv_sem=rsems.at[0 if off == 0 else 1, h_hop],
                device_id=(dev,), device_id_type=pl.DeviceIdType.MESH)
            rdma.start()
            hop.append(rdma)
        hop[0].wait_recv()
        hop[1].wait_recv()
        rdmas += hop

    for r in rdmas:
        r.wait_send()


def kernel(x, Wq, K_ext, V_ext, Wo):
    k2 = K_ext.reshape(SKV_LOC, HQ * DH)
    v2 = V_ext.reshape(SKV_LOC, HQ * DH)
    o, m, l = pl.pallas_call(
        _attn_body,
        grid=(N_QT, HQ),
        in_specs=[
            pl.BlockSpec((1, QT, D), lambda t, h: (0, t, 0)),
            pl.BlockSpec((D, DH), lambda t, h: (0, h)),
            pl.BlockSpec((SKV_LOC, DH), lambda t, h: (0, h)),
            pl.BlockSpec((SKV_LOC, DH), lambda t, h: (0, h)),
        ],
        out_specs=[
            pl.BlockSpec((QT, DH), lambda t, h: (t, h)),
            pl.BlockSpec((QT, HQ), lambda t, h: (t, 0)),
            pl.BlockSpec((QT, HQ), lambda t, h: (t, 0)),
        ],
        out_shape=[
            jax.ShapeDtypeStruct((SQ, D), jnp.float32),
            jax.ShapeDtypeStruct((SQ, HQ), jnp.float32),
            jax.ShapeDtypeStruct((SQ, HQ), jnp.float32),
        ],
    )(x, Wq, k2, v2)

    return pl.pallas_call(
        _ring_body,
        out_shape=jax.ShapeDtypeStruct((1, SQ, D), jnp.float32),
        in_specs=[pl.BlockSpec(memory_space=pltpu.VMEM)] * 4,
        out_specs=pl.BlockSpec(memory_space=pltpu.VMEM),
        scratch_shapes=[
            pltpu.VMEM((SQ, D), jnp.float32),
            pltpu.VMEM((2, SQ, HQ), jnp.float32),
            pltpu.VMEM((N_DEV - 1, HALF, D), jnp.float32),
            pltpu.VMEM((N_DEV - 1, 2, HALF, HQ), jnp.float32),
            pltpu.VMEM((N_DEV - 1, HALF, D), jnp.float32),
            pltpu.VMEM((N_DEV - 1, 2, HALF, HQ), jnp.float32),
            pltpu.SemaphoreType.DMA((2, N_DEV - 1)),
            pltpu.SemaphoreType.DMA((2, N_DEV - 1)),
            pltpu.SemaphoreType.DMA((2, N_DEV - 1)),
            pltpu.SemaphoreType.DMA((2, N_DEV - 1)),
            pltpu.SemaphoreType.DMA((2, N_DEV - 1)),
            pltpu.SemaphoreType.DMA((2, N_DEV - 1)),
        ],
        compiler_params=pltpu.CompilerParams(collective_id=0),
    )(o, m, l, Wo)
